# baseline (speedup 1.0000x reference)
import functools

import jax
import jax.numpy as jnp
from jax import lax
from jax.experimental import pallas as pl
from jax.experimental.pallas import tpu as pltpu
from jax.experimental.pallas import tpu_sc as plsc

VOCAB = 1000000
DIM = 64
BATCH = 4096
HIST = 200

B = BATCH * HIST
CHUNK = 128
SUB = 8
GROUP = CHUNK * SUB


def _make_kernel(num_workers):
    b_per_w = B // num_workers
    groups = b_per_w // GROUP
    rows_per_w = b_per_w // CHUNK

    mesh = plsc.VectorSubcoreMesh(core_axis_name="c", subcore_axis_name="s")

    @functools.partial(
        pl.kernel,
        mesh=mesh,
        out_type=jax.ShapeDtypeStruct((B, DIM), jnp.float32),
        scratch_types=[
            pltpu.VMEM((SUB, CHUNK), jnp.int32),
            pltpu.VMEM((GROUP, DIM), jnp.float32),
            pltpu.SemaphoreType.DMA,
        ],
        compiler_params=pltpu.CompilerParams(use_tc_tiling_on_sc=False),
    )
    def gather_kernel(idx_hbm, table_hbm, out_hbm, idx_v, rows_v, sem):
        num_cores = lax.axis_size("c")
        wid = lax.axis_index("s") * num_cores + lax.axis_index("c")
        row_base = wid * rows_per_w

        def body(g, carry):
            grp_row = row_base + g * SUB
            pltpu.sync_copy(idx_hbm.at[pl.ds(grp_row, SUB)], idx_v)
            copies = [
                pltpu.async_copy(
                    table_hbm.at[idx_v.at[j]],
                    rows_v.at[pl.ds(j * CHUNK, CHUNK)],
                    sem,
                )
                for j in range(SUB)
            ]
            for c in copies:
                c.wait()
            pltpu.sync_copy(rows_v, out_hbm.at[pl.ds(grp_row * CHUNK, GROUP)])
            return carry

        lax.fori_loop(0, groups, body, 0)

    return gather_kernel


def kernel(indices, table):
    info = plsc.get_sparse_core_info()
    num_workers = info.num_cores * info.num_subcores
    idx2d = indices.reshape(B // CHUNK, CHUNK)
    out = _make_kernel(num_workers)(idx2d, table)
    return out.reshape(BATCH, HIST, DIM)


# speedup vs baseline: 1.0159x; 1.0159x over previous
import functools

import jax
import jax.numpy as jnp
from jax import lax
from jax.experimental import pallas as pl
from jax.experimental.pallas import tpu as pltpu
from jax.experimental.pallas import tpu_sc as plsc

VOCAB = 1000000
DIM = 64
BATCH = 4096
HIST = 200

B = BATCH * HIST
CHUNK = 128
SUB = 4
GROUP = CHUNK * SUB


def _make_kernel(num_workers):
    b_per_w = B // num_workers
    groups = b_per_w // GROUP
    pairs = groups // 2
    rows_per_w = b_per_w // CHUNK

    mesh = plsc.VectorSubcoreMesh(core_axis_name="c", subcore_axis_name="s")

    @functools.partial(
        pl.kernel,
        mesh=mesh,
        out_type=jax.ShapeDtypeStruct((B, DIM), jnp.float32),
        scratch_types=[
            pltpu.VMEM((SUB, CHUNK), jnp.int32),
            pltpu.VMEM((SUB, CHUNK), jnp.int32),
            pltpu.VMEM((GROUP, DIM), jnp.float32),
            pltpu.VMEM((GROUP, DIM), jnp.float32),
            pltpu.SemaphoreType.DMA,
            pltpu.SemaphoreType.DMA,
            pltpu.SemaphoreType.DMA,
            pltpu.SemaphoreType.DMA,
        ],
        compiler_params=pltpu.CompilerParams(use_tc_tiling_on_sc=False),
    )
    def gather_kernel(idx_hbm, table_hbm, out_hbm, idx0, idx1, rows0, rows1,
                      sem_g0, sem_g1, sem_o0, sem_o1):
        num_cores = lax.axis_size("c")
        wid = lax.axis_index("s") * num_cores + lax.axis_index("c")
        row_base = wid * rows_per_w

        def stage_idx(g, idx_v):
            pltpu.sync_copy(idx_hbm.at[pl.ds(row_base + g * SUB, SUB)], idx_v)

        def fire_gather(idx_v, rows_v, sem):
            for j in range(SUB):
                pltpu.async_copy(
                    table_hbm.at[idx_v.at[j]],
                    rows_v.at[pl.ds(j * CHUNK, CHUNK)],
                    sem,
                )

        def wait_rows(rows_v, sem):
            pltpu.make_async_copy(out_hbm.at[pl.ds(0, GROUP)], rows_v, sem).wait()

        def fire_out(g, rows_v, sem):
            pltpu.async_copy(
                rows_v, out_hbm.at[pl.ds((row_base + g * SUB) * CHUNK, GROUP)], sem
            )

        stage_idx(0, idx0)
        fire_gather(idx0, rows0, sem_g0)

        def body(k, carry):
            g0 = 2 * k
            stage_idx(g0 + 1, idx1)

            @pl.when(k > 0)
            def _():
                wait_rows(rows1, sem_o1)

            fire_gather(idx1, rows1, sem_g1)

            wait_rows(rows0, sem_g0)
            fire_out(g0, rows0, sem_o0)

            g_next = lax.min(g0 + 2, groups - 1)
            stage_idx(g_next, idx0)
            wait_rows(rows0, sem_o0)
            fire_gather(idx0, rows0, sem_g0)

            wait_rows(rows1, sem_g1)
            fire_out(g0 + 1, rows1, sem_o1)
            return carry

        lax.fori_loop(0, pairs, body, 0)

        wait_rows(rows0, sem_g0)
        wait_rows(rows1, sem_o1)

    return gather_kernel


def kernel(indices, table):
    info = plsc.get_sparse_core_info()
    num_workers = info.num_cores * info.num_subcores
    idx2d = indices.reshape(B // CHUNK, CHUNK)
    out = _make_kernel(num_workers)(idx2d, table)
    return out.reshape(BATCH, HIST, DIM)
